# Initial kernel scaffold; baseline (speedup 1.0000x reference)
import jax
import jax.numpy as jnp
from jax.experimental import pallas as pl


def kernel(x, edge_index, batch, W1, b1, W2, b2, W3, b3, W4, b4, W5, b5):
    raise NotImplementedError("write your pallas kernel here")


# speedup vs baseline: 14.8037x; 14.8037x over previous
import functools

import jax
import jax.numpy as jnp
from jax import lax
from jax.experimental import pallas as pl
from jax.experimental.pallas import tpu as pltpu
from jax.experimental.pallas import tpu_sc as plsc

N = 10000
NUM_GRAPHS = 16
NP = 10240
ROWS_PER_TILE = NP // 16
CHUNK = 128
NW = 32
DEGW = 16

_MESH = plsc.VectorSubcoreMesh(core_axis_name="c", subcore_axis_name="s")


def _fill(ref, rows, d, value):
    vec = jnp.full((16,), value, jnp.float32)
    for r in range(rows):
        for cc in range(d // 16):
            ref[r, pl.ds(cc * 16, 16)] = vec


def _make_agg(d, n_chunks):
    n2 = n_chunks // 2

    @functools.partial(
        pl.kernel,
        mesh=_MESH,
        out_type=jax.ShapeDtypeStruct((2, NP, d), jnp.float32),
        scratch_types=[
            pltpu.VMEM((n_chunks + 2, CHUNK), jnp.int32),
            pltpu.VMEM((n_chunks + 2, CHUNK), jnp.int32),
            pltpu.VMEM((CHUNK, d), jnp.float32),
            pltpu.VMEM((CHUNK, d), jnp.float32),
            pltpu.VMEM_SHARED((NP, d), jnp.float32),
            pltpu.SemaphoreType.DMA,
            pltpu.SemaphoreType.DMA,
        ],
        compiler_params=pltpu.CompilerParams(use_tc_tiling_on_sc=False),
    )
    def agg(g_hbm, src_hbm, dst_hbm, out_hbm,
            src_v, dst_v, rows_a, rows_b, acc_sh, sem_a, sem_b):
        c = lax.axis_index("c")
        s = lax.axis_index("s")
        w = c * 16 + s
        _fill(rows_a, CHUNK, d, 0.0)
        pltpu.sync_copy(src_hbm.at[w], src_v)
        pltpu.sync_copy(dst_hbm.at[w], dst_v)
        base = s * ROWS_PER_TILE
        for i in range(ROWS_PER_TILE // CHUNK):
            pltpu.sync_copy(rows_a, acc_sh.at[pl.ds(base + i * CHUNK, CHUNK)])
        plsc.subcore_barrier()

        pltpu.async_copy(g_hbm.at[src_v.at[0]], rows_a, sem_a).wait()

        def body(j2, carry):
            j = 2 * j2
            db = pltpu.async_copy(g_hbm.at[src_v.at[j + 1]], rows_b, sem_b)
            pltpu.sync_copy(rows_a, acc_sh.at[dst_v.at[j]], add=True)
            db.wait()
            da = pltpu.async_copy(g_hbm.at[src_v.at[j + 2]], rows_a, sem_a)
            pltpu.sync_copy(rows_b, acc_sh.at[dst_v.at[j + 1]], add=True)
            da.wait()
            return carry

        lax.fori_loop(0, n2, body, 0)
        plsc.subcore_barrier()
        pltpu.sync_copy(acc_sh.at[pl.ds(base, ROWS_PER_TILE)],
                        out_hbm.at[c, pl.ds(base, ROWS_PER_TILE)])

    return agg


def _make_deg(n_chunks):
    @functools.partial(
        pl.kernel,
        mesh=_MESH,
        out_type=jax.ShapeDtypeStruct((2, NP, DEGW), jnp.float32),
        scratch_types=[
            pltpu.VMEM((n_chunks + 2, CHUNK), jnp.int32),
            pltpu.VMEM((CHUNK, DEGW), jnp.float32),
            pltpu.VMEM_SHARED((NP, DEGW), jnp.float32),
        ],
        compiler_params=pltpu.CompilerParams(use_tc_tiling_on_sc=False),
    )
    def degk(dst_hbm, out_hbm, dst_v, ones_v, acc_sh):
        c = lax.axis_index("c")
        s = lax.axis_index("s")
        w = c * 16 + s
        pltpu.sync_copy(dst_hbm.at[w], dst_v)
        _fill(ones_v, CHUNK, DEGW, 0.0)
        base = s * ROWS_PER_TILE
        for i in range(ROWS_PER_TILE // CHUNK):
            pltpu.sync_copy(ones_v, acc_sh.at[pl.ds(base + i * CHUNK, CHUNK)])
        _fill(ones_v, CHUNK, DEGW, 1.0)
        plsc.subcore_barrier()

        def body(j, carry):
            pltpu.sync_copy(ones_v, acc_sh.at[dst_v.at[j]], add=True)
            return carry

        lax.fori_loop(0, n_chunks, body, 0)
        plsc.subcore_barrier()
        pltpu.sync_copy(acc_sh.at[pl.ds(base, ROWS_PER_TILE)],
                        out_hbm.at[c, pl.ds(base, ROWS_PER_TILE)])

    return degk


def _dinv_from(deg_ref):
    deg = deg_ref[0, :N, 0:1] + deg_ref[1, :N, 0:1] + 1.0
    return lax.rsqrt(deg)


def _pad_rows(h):
    return jnp.concatenate(
        [h, jnp.zeros((NP - N, h.shape[1]), jnp.float32)], axis=0)


def _tc_first(x, degp, W1):
    def body(x_ref, deg_ref, w_ref, o_ref):
        dinv = _dinv_from(deg_ref)
        h = jnp.dot(x_ref[...], w_ref[...], preferred_element_type=jnp.float32)
        o_ref[...] = _pad_rows(h * dinv)

    return pl.pallas_call(
        body, out_shape=jax.ShapeDtypeStruct((NP, W1.shape[1]), jnp.float32),
    )(x, degp, W1)


def _tc_mid(accp, g, degp, b_prev, W, d_eff, d_out_pad):
    d_out = W.shape[1]

    def body(acc_ref, g_ref, deg_ref, b_ref, w_ref, o_ref):
        dinv = _dinv_from(deg_ref)
        tot = (acc_ref[0, :N, :d_eff] + acc_ref[1, :N, :d_eff]
               + g_ref[:N, :d_eff])
        a = jax.nn.relu(dinv * tot + b_ref[...])
        h = jnp.dot(a, w_ref[...], preferred_element_type=jnp.float32) * dinv
        if d_out_pad > d_out:
            h = jnp.concatenate(
                [h, jnp.zeros((N, d_out_pad - d_out), jnp.float32)], axis=1)
        o_ref[...] = _pad_rows(h)

    return pl.pallas_call(
        body, out_shape=jax.ShapeDtypeStruct((NP, d_out_pad), jnp.float32),
    )(accp, g, degp, b_prev, W)


def _tc_scale(accp, g, degp, b_prev, d_eff, d_out_pad):
    def body(acc_ref, g_ref, deg_ref, b_ref, o_ref):
        dinv = _dinv_from(deg_ref)
        tot = (acc_ref[0, :N, :d_eff] + acc_ref[1, :N, :d_eff]
               + g_ref[:N, :d_eff])
        a = jax.nn.relu(dinv * tot + b_ref[...])
        h = a * dinv
        if d_out_pad > d_eff:
            h = jnp.concatenate(
                [h, jnp.zeros((N, d_out_pad - d_eff), jnp.float32)], axis=1)
        o_ref[...] = _pad_rows(h)

    return pl.pallas_call(
        body, out_shape=jax.ShapeDtypeStruct((NP, d_out_pad), jnp.float32),
    )(accp, g, degp, b_prev)


def _tc_last(accp, g, degp, W5, b5, batch2d, d_eff):
    def body(acc_ref, g_ref, deg_ref, w_ref, b_ref, batch_ref, o_ref):
        dinv = _dinv_from(deg_ref)
        tot = (acc_ref[0, :N, :d_eff] + acc_ref[1, :N, :d_eff]
               + g_ref[:N, :d_eff])
        t = dinv * tot
        a = jax.nn.relu(
            jnp.dot(t, w_ref[...], preferred_element_type=jnp.float32)
            + b_ref[...])
        gids = batch_ref[...]
        onehot = (gids == lax.broadcasted_iota(jnp.int32, (1, NUM_GRAPHS), 1)
                  ).astype(jnp.float32)
        sums = lax.dot_general(onehot, a, (((0,), (0,)), ((), ())),
                               preferred_element_type=jnp.float32)
        cnt = jnp.sum(onehot, axis=0)[:, None]
        o_ref[...] = sums / jnp.maximum(cnt, 1.0)

    return pl.pallas_call(
        body, out_shape=jax.ShapeDtypeStruct((NUM_GRAPHS, W5.shape[1]), jnp.float32),
    )(accp, g, degp, W5, b5, batch2d)


def kernel(x, edge_index, batch, W1, b1, W2, b2, W3, b3, W4, b4, W5, b5):
    src = edge_index[0].astype(jnp.int32)
    dst = edge_index[1].astype(jnp.int32)
    E = src.shape[0]
    n_chunks = -(-E // (NW * CHUNK))
    if n_chunks % 2:
        n_chunks += 1
    epad = NW * n_chunks * CHUNK
    srcp = jnp.concatenate(
        [src, jnp.zeros((epad - E,), jnp.int32)]).reshape(NW, n_chunks, CHUNK)
    dstp = jnp.concatenate(
        [dst, jnp.full((epad - E,), N, jnp.int32)]).reshape(NW, n_chunks, CHUNK)
    srcp = jnp.concatenate(
        [srcp, jnp.zeros((NW, 2, CHUNK), jnp.int32)], axis=1)
    dstp = jnp.concatenate(
        [dstp, jnp.full((NW, 2, CHUNK), N, jnp.int32)], axis=1)
    batch2d = batch.astype(jnp.int32).reshape(N, 1)
    b1r, b2r, b3r, b4r, b5r = (b.reshape(1, -1) for b in (b1, b2, b3, b4, b5))

    degp = _make_deg(n_chunks)(dstp)

    agg64 = _make_agg(64, n_chunks)
    agg32 = _make_agg(32, n_chunks)
    agg16 = _make_agg(16, n_chunks)

    g1 = _tc_first(x, degp, W1)
    acc1 = agg64(g1, srcp, dstp)
    g2 = _tc_mid(acc1, g1, degp, b1r, W2, 64, 32)
    acc2 = agg32(g2, srcp, dstp)
    g3 = _tc_mid(acc2, g2, degp, b2r, W3, 32, 16)
    acc3 = agg16(g3, srcp, dstp)
    g4 = _tc_mid(acc3, g3, degp, b3r, W4, 16, 16)
    acc4 = agg16(g4, srcp, dstp)
    g5 = _tc_scale(acc4, g4, degp, b4r, 8, 16)
    acc5 = agg16(g5, srcp, dstp)
    return _tc_last(acc5, g5, degp, W5, b5r, batch2d, 8)
